# baseline (speedup 1.0000x reference)
import functools

import jax
import jax.numpy as jnp
from jax import lax
from jax.experimental import pallas as pl
from jax.experimental.pallas import tpu as pltpu
from jax.experimental.pallas import tpu_sc as plsc

B = 16384
S = 200
VOCAB = 95
VOC = 96
EMBED = 32
HIDDEN = 64
LATENT = 32

NC = 2
NS = 16
NW = NC * NS
CH = 128

_mesh = plsc.VectorSubcoreMesh(core_axis_name="c", subcore_axis_name="s")


def _make_sc_histogram(nrows, base):
    rpw = nrows // NW
    nchunk = rpw // CH

    @functools.partial(
        pl.kernel,
        mesh=_mesh,
        out_type=jax.ShapeDtypeStruct((nrows * VOC,), jnp.float32),
        scratch_types=[
            pltpu.VMEM((S, CH), jnp.int32),
            pltpu.VMEM((S, CH), jnp.int32),
            pltpu.VMEM((rpw * VOC,), jnp.float32),
            pltpu.SemaphoreType.DMA,
            pltpu.SemaphoreType.DMA,
            pltpu.SemaphoreType.DMA,
        ],
        compiler_params=pltpu.CompilerParams(needs_layout_passes=False),
    )
    def _sc_histogram(tok_hbm, cnt_hbm, tok_v0, tok_v1, cnt_v, sem0, sem1,
                      sem_out):
        wid = lax.axis_index("s") * NC + lax.axis_index("c")
        row0 = base + wid * rpw
        lanes = lax.iota(jnp.int32, 16)
        zeros = jnp.zeros((16,), jnp.float32)
        ones = jnp.ones((16,), jnp.float32)
        bufs = (tok_v0, tok_v1)
        sems = (sem0, sem1)

        in_descs = [None] * nchunk
        in_descs[0] = pltpu.async_copy(tok_hbm.at[:, pl.ds(row0, CH)],
                                       tok_v0, sem0)

        def zbody(i, carry):
            cnt_v[pl.ds(i * 16, 16)] = zeros
            return carry

        lax.fori_loop(0, rpw * VOC // 16, zbody, 0, unroll=8)

        for ch in range(nchunk):
            if ch + 1 < nchunk:
                in_descs[ch + 1] = pltpu.async_copy(
                    tok_hbm.at[:, pl.ds(row0 + (ch + 1) * CH, CH)],
                    bufs[(ch + 1) % 2], sems[(ch + 1) % 2])
            in_descs[ch].wait()
            tok_v = bufs[ch % 2]
            cnt_bases = [((ch * CH + rb * 16) + lanes) * VOC
                         for rb in range(CH // 16)]
            loc_groups = [rb * 16 + lanes for rb in range(CH // 16)]

            def sbody(s, carry):
                svec = jnp.full((16,), 0, jnp.int32) + s
                ts = [plsc.load_gather(tok_v, [svec, loc_groups[rb]])
                      for rb in range(CH // 16)]
                for rb in range(CH // 16):
                    plsc.addupdate_scatter(cnt_v, [cnt_bases[rb] + ts[rb]],
                                           ones)
                return carry

            lax.fori_loop(0, S, sbody, 0, unroll=2)

        pltpu.async_copy(cnt_v, cnt_hbm.at[pl.ds(wid * rpw * VOC, rpw * VOC)],
                         sem_out).wait()

    return _sc_histogram


HALF = B // 2
_sc_hist_h0 = _make_sc_histogram(HALF, 0)
_sc_hist_h1 = _make_sc_histogram(HALF, HALF)


def _tc_mlp_t(cnt_ref, embt_ref, w1t_ref, b1_ref, w2t_ref, b2_ref, w3t_ref,
              b3_ref, zin_ref, out_ref):
    del zin_ref
    cnts = cnt_ref[...]
    e1t = jnp.dot(w1t_ref[...], embt_ref[...],
                  preferred_element_type=jnp.float32)
    acc = lax.dot_general(e1t, cnts, (((1,), (1,)), ((), ())),
                          preferred_element_type=jnp.float32)
    sel0 = (lax.broadcasted_iota(jnp.int32, (1, VOC), 1) == 0
            ).astype(jnp.float32)
    nz0 = lax.dot_general(sel0, cnts, (((1,), (1,)), ((), ())),
                          preferred_element_type=jnp.float32)
    denom = (jnp.float32(S) - nz0) + jnp.float32(1e-8)
    h = jnp.maximum(acc / denom + b1_ref[...], 0.0)
    h = jnp.maximum(
        jnp.dot(w2t_ref[...], h, preferred_element_type=jnp.float32)
        + b2_ref[...], 0.0)
    out_ref[...] = (jnp.dot(w3t_ref[...], h,
                            preferred_element_type=jnp.float32)
                    + b3_ref[...])


BLK = 2048


def _run_mlp_t(counts, embt, W1t, b1c, W2t, b2c, W3t, b3c, zin, blk_off):
    rep = lambda i: (0, 0)
    ncols = counts.shape[0]
    in_specs = [
        pl.BlockSpec((BLK, VOC), lambda i: (i, 0)),
        pl.BlockSpec((EMBED, VOC), rep),
        pl.BlockSpec((HIDDEN, EMBED), rep),
        pl.BlockSpec((HIDDEN, 1), rep),
        pl.BlockSpec((HIDDEN, HIDDEN), rep),
        pl.BlockSpec((HIDDEN, 1), rep),
        pl.BlockSpec((LATENT, HIDDEN), rep),
        pl.BlockSpec((LATENT, 1), rep),
    ]
    args = [counts, embt, W1t, b1c, W2t, b2c, W3t, b3c]
    body = _tc_mlp_t
    aliases = {}
    if zin is not None:
        in_specs.append(pl.BlockSpec((LATENT, BLK),
                                     lambda i: (0, i + blk_off)))
        args.append(zin)
        aliases = {8: 0}
    else:
        body = functools.partial(_tc_mlp_t_noalias)
    return pl.pallas_call(
        body,
        grid=(ncols // BLK,),
        in_specs=in_specs,
        out_specs=pl.BlockSpec((LATENT, BLK), lambda i: (0, i + blk_off)),
        out_shape=jax.ShapeDtypeStruct((LATENT, B), jnp.float32),
        input_output_aliases=aliases,
    )(*args)


def _tc_mlp_t_noalias(cnt_ref, embt_ref, w1t_ref, b1_ref, w2t_ref, b2_ref,
                      w3t_ref, b3_ref, out_ref):
    _tc_mlp_t(cnt_ref, embt_ref, w1t_ref, b1_ref, w2t_ref, b2_ref, w3t_ref,
              b3_ref, None, out_ref)


def kernel(tokens, emb, W1, b1, W2, b2, W3, b3):
    tokens_t = tokens.astype(jnp.int32).T
    embt = jnp.pad(emb.astype(jnp.float32), ((0, VOC - VOCAB), (0, 0))).T
    W1t, W2t, W3t = W1.T, W2.T, W3.T
    b1c, b2c, b3c = (b1.reshape(-1, 1), b2.reshape(-1, 1), b3.reshape(-1, 1))
    c0 = _sc_hist_h0(tokens_t).reshape(HALF, VOC)
    c1 = _sc_hist_h1(tokens_t).reshape(HALF, VOC)
    zt = _run_mlp_t(c0, embt, W1t, b1c, W2t, b2c, W3t, b3c, None, 0)
    zt = _run_mlp_t(c1, embt, W1t, b1c, W2t, b2c, W3t, b3c, zt, HALF // BLK)
    return zt.T


# speedup vs baseline: 1.1299x; 1.1299x over previous
import functools

import jax
import jax.numpy as jnp
from jax import lax
from jax.experimental import pallas as pl
from jax.experimental.pallas import tpu as pltpu
from jax.experimental.pallas import tpu_sc as plsc

B = 16384
S = 200
VOCAB = 95
VOC = 96
EMBED = 32
HIDDEN = 64
LATENT = 32

NC = 2
NS = 16
NW = NC * NS
CH = 128

_mesh = plsc.VectorSubcoreMesh(core_axis_name="c", subcore_axis_name="s")


def _make_sc_histogram(nrows, base):
    rpw = nrows // NW
    nchunk = rpw // CH

    @functools.partial(
        pl.kernel,
        mesh=_mesh,
        out_type=jax.ShapeDtypeStruct((VOC, nrows), jnp.float32),
        scratch_types=[
            pltpu.VMEM((S, CH), jnp.int32),
            pltpu.VMEM((S, CH), jnp.int32),
            pltpu.VMEM((VOC, rpw), jnp.float32),
            pltpu.SemaphoreType.DMA,
            pltpu.SemaphoreType.DMA,
            pltpu.SemaphoreType.DMA,
        ],
        compiler_params=pltpu.CompilerParams(needs_layout_passes=False),
    )
    def _sc_histogram(tok_hbm, cnt_hbm, tok_v0, tok_v1, cnt_v, sem0, sem1,
                      sem_out):
        wid = lax.axis_index("s") * NC + lax.axis_index("c")
        row0 = base + wid * rpw
        lanes = lax.iota(jnp.int32, 16)
        zeros = jnp.zeros((16,), jnp.float32)
        ones = jnp.ones((16,), jnp.float32)
        bufs = (tok_v0, tok_v1)
        sems = (sem0, sem1)

        in_descs = [None] * nchunk
        in_descs[0] = pltpu.async_copy(tok_hbm.at[:, pl.ds(row0, CH)],
                                       tok_v0, sem0)

        def zbody(i, carry):
            for c in range(rpw // 16):
                cnt_v[i, pl.ds(c * 16, 16)] = zeros
            return carry

        lax.fori_loop(0, VOC, zbody, 0, unroll=2)

        for ch in range(nchunk):
            if ch + 1 < nchunk:
                in_descs[ch + 1] = pltpu.async_copy(
                    tok_hbm.at[:, pl.ds(row0 + (ch + 1) * CH, CH)],
                    bufs[(ch + 1) % 2], sems[(ch + 1) % 2])
            in_descs[ch].wait()
            tok_v = bufs[ch % 2]
            row_groups = [(ch * CH + rb * 16) + lanes
                          for rb in range(CH // 16)]
            loc_groups = [rb * 16 + lanes for rb in range(CH // 16)]

            def sbody(s, carry):
                svec = jnp.full((16,), 0, jnp.int32) + s
                ts = [plsc.load_gather(tok_v, [svec, loc_groups[rb]])
                      for rb in range(CH // 16)]
                for rb in range(CH // 16):
                    plsc.addupdate_scatter(cnt_v, [ts[rb], row_groups[rb]],
                                           ones)
                return carry

            lax.fori_loop(0, S, sbody, 0, unroll=2)

        pltpu.async_copy(cnt_v, cnt_hbm.at[:, pl.ds(wid * rpw, rpw)],
                         sem_out).wait()

    return _sc_histogram


HALF = B // 2
_sc_hist_h0 = _make_sc_histogram(HALF, 0)
_sc_hist_h1 = _make_sc_histogram(HALF, HALF)


def _tc_mlp_t(cnt_ref, embt_ref, w1t_ref, b1_ref, w2t_ref, b2_ref, w3t_ref,
              b3_ref, zin_ref, out_ref):
    del zin_ref
    cnts = cnt_ref[...]
    e1t = jnp.dot(w1t_ref[...], embt_ref[...],
                  preferred_element_type=jnp.float32)
    acc = jnp.dot(e1t, cnts, preferred_element_type=jnp.float32)
    denom = (jnp.float32(S) - cnts[0:1, :]) + jnp.float32(1e-8)
    h = jnp.maximum(acc / denom + b1_ref[...], 0.0)
    h = jnp.maximum(
        jnp.dot(w2t_ref[...], h, preferred_element_type=jnp.float32)
        + b2_ref[...], 0.0)
    out_ref[...] = (jnp.dot(w3t_ref[...], h,
                            preferred_element_type=jnp.float32)
                    + b3_ref[...])


BLK = 2048


def _run_mlp_t(counts, embt, W1t, b1c, W2t, b2c, W3t, b3c, zin, blk_off):
    rep = lambda i: (0, 0)
    ncols = counts.shape[1]
    in_specs = [
        pl.BlockSpec((VOC, BLK), lambda i: (0, i)),
        pl.BlockSpec((EMBED, VOC), rep),
        pl.BlockSpec((HIDDEN, EMBED), rep),
        pl.BlockSpec((HIDDEN, 1), rep),
        pl.BlockSpec((HIDDEN, HIDDEN), rep),
        pl.BlockSpec((HIDDEN, 1), rep),
        pl.BlockSpec((LATENT, HIDDEN), rep),
        pl.BlockSpec((LATENT, 1), rep),
    ]
    args = [counts, embt, W1t, b1c, W2t, b2c, W3t, b3c]
    body = _tc_mlp_t
    aliases = {}
    if zin is not None:
        in_specs.append(pl.BlockSpec((LATENT, BLK),
                                     lambda i: (0, i + blk_off)))
        args.append(zin)
        aliases = {8: 0}
    else:
        body = functools.partial(_tc_mlp_t_noalias)
    return pl.pallas_call(
        body,
        grid=(ncols // BLK,),
        in_specs=in_specs,
        out_specs=pl.BlockSpec((LATENT, BLK), lambda i: (0, i + blk_off)),
        out_shape=jax.ShapeDtypeStruct((LATENT, B), jnp.float32),
        input_output_aliases=aliases,
    )(*args)


def _tc_mlp_t_noalias(cnt_ref, embt_ref, w1t_ref, b1_ref, w2t_ref, b2_ref,
                      w3t_ref, b3_ref, out_ref):
    _tc_mlp_t(cnt_ref, embt_ref, w1t_ref, b1_ref, w2t_ref, b2_ref, w3t_ref,
              b3_ref, None, out_ref)


def kernel(tokens, emb, W1, b1, W2, b2, W3, b3):
    tokens_t = tokens.astype(jnp.int32).T
    embt = jnp.pad(emb.astype(jnp.float32), ((0, VOC - VOCAB), (0, 0))).T
    W1t, W2t, W3t = W1.T, W2.T, W3.T
    b1c, b2c, b3c = (b1.reshape(-1, 1), b2.reshape(-1, 1), b3.reshape(-1, 1))
    c0 = _sc_hist_h0(tokens_t)
    c1 = _sc_hist_h1(tokens_t)
    zt = _run_mlp_t(c0, embt, W1t, b1c, W2t, b2c, W3t, b3c, None, 0)
    zt = _run_mlp_t(c1, embt, W1t, b1c, W2t, b2c, W3t, b3c, zt, HALF // BLK)
    return zt.T
